# Initial kernel scaffold; baseline (speedup 1.0000x reference)
import functools

import jax
import jax.numpy as jnp
from jax import lax
from jax.experimental import pallas as pl
from jax.experimental.pallas import tpu as pltpu
from jax.experimental.pallas import tpu_sc as plsc

N = 50000
E = 800000
D_IN = 64
D_HID = 128

NP = 50176
TPT = NP // 16
EP = 819200
EROWS = EP // 128
RPT = EROWS // 16
KB = 40
NB_OUT = RPT // KB
DROWS = EROWS // 2
DRPT = DROWS // 16
ZR = 784

_mesh = plsc.VectorSubcoreMesh(core_axis_name="c", subcore_axis_name="s")


def _fill(ref, rows, val):
  v = jnp.full((16,), val, jnp.float32)
  if len(ref.shape) == 1:
    @pl.loop(0, rows // 16)
    def _(i):
      ref[pl.ds(i * 16, 16)] = v
  else:
    @pl.loop(0, rows)
    def _(i):
      ref[i, 0:16] = v
      ref[i, 16:32] = v


@functools.partial(
    pl.kernel,
    out_type=jax.ShapeDtypeStruct((2, NP), jnp.float32),
    mesh=_mesh,
    scratch_types=[
        pltpu.VMEM((KB, 128), jnp.int32),
        pltpu.VMEM((128,), jnp.float32),
        pltpu.VMEM((ZR,), jnp.float32),
        pltpu.VMEM_SHARED((NP,), jnp.float32),
    ],
)
def _deg_kernel(dst_hbm, deg_out, idx_d, ones_v, zb, deg_sp):
  c = lax.axis_index("c")
  s = lax.axis_index("s")
  _fill(ones_v, 128, 1.0)
  _fill(zb, ZR, 0.0)
  for z in range(4):
    pltpu.sync_copy(zb, deg_sp.at[pl.ds(s * TPT + z * ZR, ZR)])
  plsc.subcore_barrier()

  @pl.loop(0, DRPT // KB)
  def _(kb):
    br = c * DROWS + s * DRPT + kb * KB
    pltpu.sync_copy(dst_hbm.at[pl.ds(br, KB)], idx_d)

    @pl.loop(0, KB)
    def _(j):
      pltpu.sync_copy(ones_v, deg_sp.at[idx_d.at[j]], add=True)

  plsc.subcore_barrier()
  pltpu.sync_copy(deg_sp.at[pl.ds(s * TPT, TPT)],
                  deg_out.at[c, pl.ds(s * TPT, TPT)])


@functools.partial(
    pl.kernel,
    out_type=jax.ShapeDtypeStruct((4, NP, 32), jnp.float32),
    mesh=_mesh,
    scratch_types=[
        pltpu.VMEM((KB, 128), jnp.int32),
        pltpu.VMEM((KB, 128), jnp.int32),
        pltpu.VMEM((128, 32), jnp.float32),
        pltpu.VMEM((ZR, 32), jnp.float32),
        pltpu.VMEM_SHARED((NP, 32), jnp.float32),
        pltpu.SemaphoreType.DMA,
    ],
)
def _scatter_kernel(src_hbm, dst_hbm, hs_hbm, acc_out,
                    idx_s, idx_d, rows, zbuf, acc_sp, sem):
  c = lax.axis_index("c")
  s = lax.axis_index("s")
  _fill(zbuf, ZR, 0.0)
  for p in range(2):
    chunk = c * 2 + p
    off = chunk * N
    for z in range(4):
      pltpu.sync_copy(zbuf, acc_sp.at[pl.ds(s * TPT + z * ZR, ZR)])
    plsc.subcore_barrier()

    @pl.loop(0, NB_OUT)
    def _(kb):
      br = s * RPT + kb * KB
      pltpu.sync_copy(src_hbm.at[pl.ds(br, KB)], idx_s)
      pltpu.sync_copy(dst_hbm.at[pl.ds(br, KB)], idx_d)

      @pl.loop(0, KB)
      def _(j):
        for m in range(8):
          idx_s[j, pl.ds(m * 16, 16)] = idx_s[j, pl.ds(m * 16, 16)] + off
        pltpu.async_copy(hs_hbm.at[idx_s.at[j]], rows, sem).wait()
        pltpu.sync_copy(rows, acc_sp.at[idx_d.at[j]], add=True)

    plsc.subcore_barrier()
    pltpu.sync_copy(acc_sp.at[pl.ds(s * TPT, TPT)],
                    acc_out.at[chunk, pl.ds(s * TPT, TPT)])
    plsc.subcore_barrier()


BN = 1000
NBLK = N // BN
_EPS = 1e-5


def _dinv_of(dpt_blk):
  deg = dpt_blk[:, 0:1] + dpt_blk[:, 1:2] + 1.0
  return lax.rsqrt(deg)


def _phase_b(x_ref, w1_ref, dpt_ref, out_ref):
  h = jnp.dot(x_ref[...], w1_ref[...], preferred_element_type=jnp.float32,
              precision=lax.Precision.HIGHEST)
  hs = h * _dinv_of(dpt_ref[...])
  for cc in range(4):
    out_ref[cc] = hs[:, cc * 32:(cc + 1) * 32]


def _phase_d(acc_ref, hs1_ref, dpt_ref, b1_ref, g1_ref, be1_ref, w2_ref,
             out_ref):
  dinv = _dinv_of(dpt_ref[...])
  o = [dinv * (acc_ref[cc] + hs1_ref[cc]) + b1_ref[0, cc * 32:(cc + 1) * 32]
       for cc in range(4)]
  mu = sum(jnp.sum(oc, axis=1, keepdims=True) for oc in o) * (1.0 / 128.0)
  d = [oc - mu for oc in o]
  var = sum(jnp.sum(dc * dc, axis=1, keepdims=True) for dc in d) * (1.0 / 128.0)
  rstd = lax.rsqrt(var + _EPS)
  h2 = None
  for cc in range(4):
    sl = slice(cc * 32, (cc + 1) * 32)
    y = jnp.maximum(d[cc] * rstd * g1_ref[0, sl] + be1_ref[0, sl], 0.0)
    t = jnp.dot(y, w2_ref[sl, :], preferred_element_type=jnp.float32,
                precision=lax.Precision.HIGHEST)
    h2 = t if h2 is None else h2 + t
  hs2 = h2 * dinv
  for cc in range(4):
    out_ref[cc] = hs2[:, cc * 32:(cc + 1) * 32]


def _phase_f(acc_ref, hs2_ref, dpt_ref, b2_ref, g2_ref, be2_ref, out_ref):
  i = pl.program_id(0)
  dinv = _dinv_of(dpt_ref[...])
  o = [dinv * (acc_ref[cc] + hs2_ref[cc]) + b2_ref[0, cc * 32:(cc + 1) * 32]
       for cc in range(4)]
  mu = sum(jnp.sum(oc, axis=1, keepdims=True) for oc in o) * (1.0 / 128.0)
  d = [oc - mu for oc in o]
  var = sum(jnp.sum(dc * dc, axis=1, keepdims=True) for dc in d) * (1.0 / 128.0)
  rstd = lax.rsqrt(var + _EPS)
  parts = []
  for cc in range(4):
    sl = slice(cc * 32, (cc + 1) * 32)
    z = d[cc] * rstd * g2_ref[0, sl] + be2_ref[0, sl]
    parts.append(jnp.sum(z, axis=0, keepdims=True))
  p = jnp.concatenate(parts, axis=0)

  @pl.when(i == 0)
  def _():
    out_ref[...] = jnp.zeros((4, 32), jnp.float32)

  out_ref[...] += p

  @pl.when(i == NBLK - 1)
  def _():
    out_ref[...] = out_ref[...] * (1.0 / N)


@jax.jit
def kernel(x, edge_index, W1, b1, ln1_w, ln1_b, W2, b2, ln2_w, ln2_b):
  src = edge_index[0].astype(jnp.int32)
  dst = edge_index[1].astype(jnp.int32)
  pad = EP - E
  pad_idx = jnp.arange(pad, dtype=jnp.int32)
  src2 = jnp.concatenate([src, pad_idx % 1024]).reshape(EROWS, 128)
  dst2 = jnp.concatenate([dst, N + 16 + pad_idx % 128]).reshape(EROWS, 128)

  deg_parts = _deg_kernel(dst2)
  dpt = jnp.swapaxes(deg_parts, 0, 1)[:N]

  b1r, g1r, be1r = b1.reshape(1, 128), ln1_w.reshape(1, 128), ln1_b.reshape(1, 128)
  b2r, g2r, be2r = b2.reshape(1, 128), ln2_w.reshape(1, 128), ln2_b.reshape(1, 128)

  hs1 = pl.pallas_call(
      _phase_b,
      grid=(NBLK,),
      in_specs=[
          pl.BlockSpec((BN, D_IN), lambda i: (i, 0)),
          pl.BlockSpec((D_IN, D_HID), lambda i: (0, 0)),
          pl.BlockSpec((BN, 2), lambda i: (i, 0)),
      ],
      out_specs=pl.BlockSpec((4, BN, 32), lambda i: (0, i, 0)),
      out_shape=jax.ShapeDtypeStruct((4, N, 32), jnp.float32),
  )(x, W1, dpt)

  acc1 = _scatter_kernel(src2, dst2, hs1.reshape(4 * N, 32))

  hs2 = pl.pallas_call(
      _phase_d,
      grid=(NBLK,),
      in_specs=[
          pl.BlockSpec((4, BN, 32), lambda i: (0, i, 0)),
          pl.BlockSpec((4, BN, 32), lambda i: (0, i, 0)),
          pl.BlockSpec((BN, 2), lambda i: (i, 0)),
          pl.BlockSpec((1, D_HID), lambda i: (0, 0)),
          pl.BlockSpec((1, D_HID), lambda i: (0, 0)),
          pl.BlockSpec((1, D_HID), lambda i: (0, 0)),
          pl.BlockSpec((D_HID, D_HID), lambda i: (0, 0)),
      ],
      out_specs=pl.BlockSpec((4, BN, 32), lambda i: (0, i, 0)),
      out_shape=jax.ShapeDtypeStruct((4, N, 32), jnp.float32),
  )(acc1, hs1, dpt, b1r, g1r, be1r, W2)

  acc2 = _scatter_kernel(src2, dst2, hs2.reshape(4 * N, 32))

  m = pl.pallas_call(
      _phase_f,
      grid=(NBLK,),
      in_specs=[
          pl.BlockSpec((4, BN, 32), lambda i: (0, i, 0)),
          pl.BlockSpec((4, BN, 32), lambda i: (0, i, 0)),
          pl.BlockSpec((BN, 2), lambda i: (i, 0)),
          pl.BlockSpec((1, D_HID), lambda i: (0, 0)),
          pl.BlockSpec((1, D_HID), lambda i: (0, 0)),
          pl.BlockSpec((1, D_HID), lambda i: (0, 0)),
      ],
      out_specs=pl.BlockSpec((4, 32), lambda i: (0, 0)),
      out_shape=jax.ShapeDtypeStruct((4, 32), jnp.float32),
  )(acc2, hs2, dpt, b2r, g2r, be2r)

  return m.reshape(1, 128)


# speedup vs baseline: 9.7310x; 9.7310x over previous
import functools

import jax
import jax.numpy as jnp
from jax import lax
from jax.experimental import pallas as pl
from jax.experimental.pallas import tpu as pltpu
from jax.experimental.pallas import tpu_sc as plsc

N = 50000
E = 800000
D_IN = 64
D_HID = 128

NP = 50176
TPT = NP // 16
EP = 819200
EROWS = EP // 128
RPT = EROWS // 16
KB = 40
NB_OUT = RPT // KB
DROWS = EROWS // 2
DRPT = DROWS // 16
ZR = 784

_mesh = plsc.VectorSubcoreMesh(core_axis_name="c", subcore_axis_name="s")


def _fill(ref, rows, val):
  v = jnp.full((16,), val, jnp.float32)
  if len(ref.shape) == 1:
    @pl.loop(0, rows // 16)
    def _(i):
      ref[pl.ds(i * 16, 16)] = v
  else:
    @pl.loop(0, rows)
    def _(i):
      ref[i, 0:16] = v
      ref[i, 16:32] = v


@functools.partial(
    pl.kernel,
    out_type=jax.ShapeDtypeStruct((2 * NP,), jnp.float32),
    mesh=_mesh,
    scratch_types=[
        pltpu.VMEM((KB, 128), jnp.int32),
        pltpu.VMEM((128,), jnp.float32),
        pltpu.VMEM((ZR,), jnp.float32),
        pltpu.VMEM((TPT,), jnp.float32),
        pltpu.VMEM_SHARED((NP,), jnp.float32),
    ],
)
def _deg_kernel(dst_hbm, deg_out, idx_d, ones_v, zb, fb, deg_sp):
  c = lax.axis_index("c")
  s = lax.axis_index("s")
  _fill(ones_v, 128, 1.0)
  _fill(zb, ZR, 0.0)
  for z in range(4):
    pltpu.sync_copy(zb, deg_sp.at[pl.ds(s * TPT + z * ZR, ZR)])
  plsc.subcore_barrier()

  @pl.loop(0, DRPT // KB)
  def _(kb):
    br = c * DROWS + s * DRPT + kb * KB
    pltpu.sync_copy(dst_hbm.at[pl.ds(br, KB)], idx_d)

    @pl.loop(0, KB)
    def _(j):
      pltpu.sync_copy(ones_v, deg_sp.at[idx_d.at[j]], add=True)

  plsc.subcore_barrier()
  pltpu.sync_copy(deg_sp.at[pl.ds(s * TPT, TPT)], fb)
  pltpu.sync_copy(fb, deg_out.at[pl.ds(c * NP + s * TPT, TPT)])


@functools.partial(
    pl.kernel,
    out_type=jax.ShapeDtypeStruct((4, NP, 32), jnp.float32),
    mesh=_mesh,
    scratch_types=[
        pltpu.VMEM((KB, 128), jnp.int32),
        pltpu.VMEM((KB, 128), jnp.int32),
        pltpu.VMEM((128, 32), jnp.float32),
        pltpu.VMEM((TPT // 16, 32), jnp.float32),
        pltpu.VMEM_SHARED((NP, 32), jnp.float32),
        pltpu.SemaphoreType.DMA,
    ],
    compiler_params=pltpu.CompilerParams(use_tc_tiling_on_sc=False),
)
def _scatter_kernel(src_hbm, dst_hbm, hs_hbm, acc_out,
                    idx_s, idx_d, rows, zf, acc_sp, sem):
  c = lax.axis_index("c")
  s = lax.axis_index("s")
  zrows = TPT // 16
  for p in range(2):
    chunk = c * 2 + p
    off = chunk * N
    _fill(zf, zrows, 0.0)
    for z in range(16):
      pltpu.sync_copy(zf, acc_sp.at[pl.ds(s * TPT + z * zrows, zrows)])
    plsc.subcore_barrier()

    @pl.loop(0, NB_OUT)
    def _(kb):
      br = s * RPT + kb * KB
      pltpu.sync_copy(src_hbm.at[pl.ds(br, KB)], idx_s)
      pltpu.sync_copy(dst_hbm.at[pl.ds(br, KB)], idx_d)

      @pl.loop(0, KB)
      def _(j):
        for m in range(8):
          idx_s[j, pl.ds(m * 16, 16)] = idx_s[j, pl.ds(m * 16, 16)] + off
        pltpu.async_copy(hs_hbm.at[idx_s.at[j]], rows, sem).wait()
        pltpu.sync_copy(rows, acc_sp.at[idx_d.at[j]], add=True)

    plsc.subcore_barrier()
    for z in range(16):
      pltpu.sync_copy(acc_sp.at[pl.ds(s * TPT + z * zrows, zrows)], zf)
      pltpu.sync_copy(zf, acc_out.at[chunk, pl.ds(s * TPT + z * zrows, zrows)])
    plsc.subcore_barrier()


BN = 1000
NBLK = N // BN
_EPS = 1e-5


def _dinv_of(dpt_blk):
  deg = dpt_blk[:, 0:1] + dpt_blk[:, 1:2] + 1.0
  return lax.rsqrt(deg)


def _phase_b(x_ref, w1_ref, dpt_ref, out_ref):
  h = jnp.dot(x_ref[...], w1_ref[...], preferred_element_type=jnp.float32,
              precision=lax.Precision.HIGHEST)
  hs = h * _dinv_of(dpt_ref[...])
  for cc in range(4):
    out_ref[cc] = hs[:, cc * 32:(cc + 1) * 32]


def _phase_d(acc_ref, hs1_ref, dpt_ref, b1_ref, g1_ref, be1_ref, w2_ref,
             out_ref):
  dinv = _dinv_of(dpt_ref[...])
  o = [dinv * (acc_ref[cc] + hs1_ref[cc]) + b1_ref[0, cc * 32:(cc + 1) * 32]
       for cc in range(4)]
  mu = sum(jnp.sum(oc, axis=1, keepdims=True) for oc in o) * (1.0 / 128.0)
  d = [oc - mu for oc in o]
  var = sum(jnp.sum(dc * dc, axis=1, keepdims=True) for dc in d) * (1.0 / 128.0)
  rstd = lax.rsqrt(var + _EPS)
  h2 = None
  for cc in range(4):
    sl = slice(cc * 32, (cc + 1) * 32)
    y = jnp.maximum(d[cc] * rstd * g1_ref[0, sl] + be1_ref[0, sl], 0.0)
    t = jnp.dot(y, w2_ref[sl, :], preferred_element_type=jnp.float32,
                precision=lax.Precision.HIGHEST)
    h2 = t if h2 is None else h2 + t
  hs2 = h2 * dinv
  for cc in range(4):
    out_ref[cc] = hs2[:, cc * 32:(cc + 1) * 32]


def _phase_f(acc_ref, hs2_ref, dpt_ref, b2_ref, g2_ref, be2_ref, out_ref):
  i = pl.program_id(0)
  dinv = _dinv_of(dpt_ref[...])
  o = [dinv * (acc_ref[cc] + hs2_ref[cc]) + b2_ref[0, cc * 32:(cc + 1) * 32]
       for cc in range(4)]
  mu = sum(jnp.sum(oc, axis=1, keepdims=True) for oc in o) * (1.0 / 128.0)
  d = [oc - mu for oc in o]
  var = sum(jnp.sum(dc * dc, axis=1, keepdims=True) for dc in d) * (1.0 / 128.0)
  rstd = lax.rsqrt(var + _EPS)
  parts = []
  for cc in range(4):
    sl = slice(cc * 32, (cc + 1) * 32)
    z = d[cc] * rstd * g2_ref[0, sl] + be2_ref[0, sl]
    parts.append(jnp.sum(z, axis=0, keepdims=True))
  p = jnp.concatenate(parts, axis=0)

  @pl.when(i == 0)
  def _():
    out_ref[...] = jnp.zeros((4, 32), jnp.float32)

  out_ref[...] += p

  @pl.when(i == NBLK - 1)
  def _():
    out_ref[...] = out_ref[...] * (1.0 / N)


@jax.jit
def kernel(x, edge_index, W1, b1, ln1_w, ln1_b, W2, b2, ln2_w, ln2_b):
  src = edge_index[0].astype(jnp.int32)
  dst = edge_index[1].astype(jnp.int32)
  pad = EP - E
  pad_idx = jnp.arange(pad, dtype=jnp.int32)
  src2 = jnp.concatenate([src, pad_idx % 1024]).reshape(EROWS, 128)
  dst2 = jnp.concatenate([dst, N + 16 + pad_idx % 128]).reshape(EROWS, 128)

  deg_parts = _deg_kernel(dst2).reshape(2, NP)
  dpt = jnp.swapaxes(deg_parts, 0, 1)[:N]

  b1r, g1r, be1r = b1.reshape(1, 128), ln1_w.reshape(1, 128), ln1_b.reshape(1, 128)
  b2r, g2r, be2r = b2.reshape(1, 128), ln2_w.reshape(1, 128), ln2_b.reshape(1, 128)

  hs1 = pl.pallas_call(
      _phase_b,
      grid=(NBLK,),
      in_specs=[
          pl.BlockSpec((BN, D_IN), lambda i: (i, 0)),
          pl.BlockSpec((D_IN, D_HID), lambda i: (0, 0)),
          pl.BlockSpec((BN, 2), lambda i: (i, 0)),
      ],
      out_specs=pl.BlockSpec((4, BN, 32), lambda i: (0, i, 0)),
      out_shape=jax.ShapeDtypeStruct((4, N, 32), jnp.float32),
  )(x, W1, dpt)

  acc1 = _scatter_kernel(src2, dst2, hs1.reshape(4 * N, 32))

  hs2 = pl.pallas_call(
      _phase_d,
      grid=(NBLK,),
      in_specs=[
          pl.BlockSpec((4, BN, 32), lambda i: (0, i, 0)),
          pl.BlockSpec((4, BN, 32), lambda i: (0, i, 0)),
          pl.BlockSpec((BN, 2), lambda i: (i, 0)),
          pl.BlockSpec((1, D_HID), lambda i: (0, 0)),
          pl.BlockSpec((1, D_HID), lambda i: (0, 0)),
          pl.BlockSpec((1, D_HID), lambda i: (0, 0)),
          pl.BlockSpec((D_HID, D_HID), lambda i: (0, 0)),
      ],
      out_specs=pl.BlockSpec((4, BN, 32), lambda i: (0, i, 0)),
      out_shape=jax.ShapeDtypeStruct((4, N, 32), jnp.float32),
  )(acc1, hs1, dpt, b1r, g1r, be1r, W2)

  acc2 = _scatter_kernel(src2, dst2, hs2.reshape(4 * N, 32))

  m = pl.pallas_call(
      _phase_f,
      grid=(NBLK,),
      in_specs=[
          pl.BlockSpec((4, BN, 32), lambda i: (0, i, 0)),
          pl.BlockSpec((4, BN, 32), lambda i: (0, i, 0)),
          pl.BlockSpec((BN, 2), lambda i: (i, 0)),
          pl.BlockSpec((1, D_HID), lambda i: (0, 0)),
          pl.BlockSpec((1, D_HID), lambda i: (0, 0)),
          pl.BlockSpec((1, D_HID), lambda i: (0, 0)),
      ],
      out_specs=pl.BlockSpec((4, 32), lambda i: (0, 0)),
      out_shape=jax.ShapeDtypeStruct((4, 32), jnp.float32),
  )(acc2, hs2, dpt, b2r, g2r, be2r)

  return m.reshape(1, 128)


# speedup vs baseline: 11.0916x; 1.1398x over previous
import functools

import jax
import jax.numpy as jnp
from jax import lax
from jax.experimental import pallas as pl
from jax.experimental.pallas import tpu as pltpu
from jax.experimental.pallas import tpu_sc as plsc

N = 50000
E = 800000
D_IN = 64
D_HID = 128

NP = 50176
TPT = NP // 16
EP = 819200
EROWS = EP // 128
RPT = EROWS // 16
KB = 40
NB_OUT = RPT // KB
DROWS = EROWS // 2
DRPT = DROWS // 16
ZR = 784

_mesh = plsc.VectorSubcoreMesh(core_axis_name="c", subcore_axis_name="s")


def _fill(ref, rows, val):
  v = jnp.full((16,), val, jnp.float32)
  if len(ref.shape) == 1:
    @pl.loop(0, rows // 16)
    def _(i):
      ref[pl.ds(i * 16, 16)] = v
  else:
    @pl.loop(0, rows)
    def _(i):
      ref[i, 0:16] = v
      ref[i, 16:32] = v


@functools.partial(
    pl.kernel,
    out_type=jax.ShapeDtypeStruct((2 * NP,), jnp.float32),
    mesh=_mesh,
    scratch_types=[
        pltpu.VMEM((KB, 128), jnp.int32),
        pltpu.VMEM((128,), jnp.float32),
        pltpu.VMEM((ZR,), jnp.float32),
        pltpu.VMEM((TPT,), jnp.float32),
        pltpu.VMEM_SHARED((NP,), jnp.float32),
    ],
)
def _deg_kernel(dst_hbm, deg_out, idx_d, ones_v, zb, fb, deg_sp):
  c = lax.axis_index("c")
  s = lax.axis_index("s")
  _fill(ones_v, 128, 1.0)
  _fill(zb, ZR, 0.0)
  for z in range(4):
    pltpu.sync_copy(zb, deg_sp.at[pl.ds(s * TPT + z * ZR, ZR)])
  plsc.subcore_barrier()

  @pl.loop(0, DRPT // KB)
  def _(kb):
    br = c * DROWS + s * DRPT + kb * KB
    pltpu.sync_copy(dst_hbm.at[pl.ds(br, KB)], idx_d)

    @pl.loop(0, KB)
    def _(j):
      pltpu.sync_copy(ones_v, deg_sp.at[idx_d.at[j]], add=True)

  plsc.subcore_barrier()
  pltpu.sync_copy(deg_sp.at[pl.ds(s * TPT, TPT)], fb)
  pltpu.sync_copy(fb, deg_out.at[pl.ds(c * NP + s * TPT, TPT)])


@functools.partial(
    pl.kernel,
    out_type=jax.ShapeDtypeStruct((4, NP, 32), jnp.float32),
    mesh=_mesh,
    scratch_types=[
        pltpu.VMEM((KB, 128), jnp.int32),
        pltpu.VMEM((KB, 128), jnp.int32),
        pltpu.VMEM((2, 128, 32), jnp.float32),
        pltpu.VMEM((TPT // 16, 32), jnp.float32),
        pltpu.VMEM_SHARED((NP, 32), jnp.float32),
        pltpu.SemaphoreType.DMA,
    ],
    compiler_params=pltpu.CompilerParams(use_tc_tiling_on_sc=False),
)
def _scatter_kernel(src_hbm, dst_hbm, hs_hbm, acc_out,
                    idx_s, idx_d, rows, zf, acc_sp, sem):
  c = lax.axis_index("c")
  s = lax.axis_index("s")
  zrows = TPT // 16
  for p in range(2):
    chunk = c * 2 + p
    off = chunk * N
    _fill(zf, zrows, 0.0)
    for z in range(16):
      pltpu.sync_copy(zf, acc_sp.at[pl.ds(s * TPT + z * zrows, zrows)])
    plsc.subcore_barrier()

    @pl.loop(0, NB_OUT)
    def _(kb):
      br = s * RPT + kb * KB
      pltpu.sync_copy(src_hbm.at[pl.ds(br, KB)], idx_s)
      pltpu.sync_copy(dst_hbm.at[pl.ds(br, KB)], idx_d)

      @pl.loop(0, KB)
      def _(j):
        for m in range(8):
          idx_s[j, pl.ds(m * 16, 16)] = idx_s[j, pl.ds(m * 16, 16)] + off

      pltpu.async_copy(hs_hbm.at[idx_s.at[0]], rows.at[0], sem)

      @pl.loop(0, KB)
      def _(j):
        par = lax.rem(j, 2)
        pltpu.make_async_copy(hs_hbm.at[idx_s.at[j]], rows.at[par], sem).wait()

        @pl.when(j < KB - 1)
        def _():
          pltpu.async_copy(hs_hbm.at[idx_s.at[j + 1]], rows.at[1 - par], sem)

        pltpu.sync_copy(rows.at[par], acc_sp.at[idx_d.at[j]], add=True)

    plsc.subcore_barrier()
    for z in range(16):
      pltpu.sync_copy(acc_sp.at[pl.ds(s * TPT + z * zrows, zrows)], zf)
      pltpu.sync_copy(zf, acc_out.at[chunk, pl.ds(s * TPT + z * zrows, zrows)])
    plsc.subcore_barrier()


BN = 1000
NBLK = N // BN
_EPS = 1e-5


def _dinv_of(dpt_blk):
  deg = dpt_blk[:, 0:1] + dpt_blk[:, 1:2] + 1.0
  return lax.rsqrt(deg)


def _phase_b(x_ref, w1_ref, dpt_ref, out_ref):
  h = jnp.dot(x_ref[...], w1_ref[...], preferred_element_type=jnp.float32,
              precision=lax.Precision.HIGHEST)
  hs = h * _dinv_of(dpt_ref[...])
  for cc in range(4):
    out_ref[cc] = hs[:, cc * 32:(cc + 1) * 32]


def _phase_d(acc_ref, hs1_ref, dpt_ref, b1_ref, g1_ref, be1_ref, w2_ref,
             out_ref):
  dinv = _dinv_of(dpt_ref[...])
  o = [dinv * (acc_ref[cc] + hs1_ref[cc]) + b1_ref[0, cc * 32:(cc + 1) * 32]
       for cc in range(4)]
  mu = sum(jnp.sum(oc, axis=1, keepdims=True) for oc in o) * (1.0 / 128.0)
  d = [oc - mu for oc in o]
  var = sum(jnp.sum(dc * dc, axis=1, keepdims=True) for dc in d) * (1.0 / 128.0)
  rstd = lax.rsqrt(var + _EPS)
  h2 = None
  for cc in range(4):
    sl = slice(cc * 32, (cc + 1) * 32)
    y = jnp.maximum(d[cc] * rstd * g1_ref[0, sl] + be1_ref[0, sl], 0.0)
    t = jnp.dot(y, w2_ref[sl, :], preferred_element_type=jnp.float32,
                precision=lax.Precision.HIGHEST)
    h2 = t if h2 is None else h2 + t
  hs2 = h2 * dinv
  for cc in range(4):
    out_ref[cc] = hs2[:, cc * 32:(cc + 1) * 32]


def _phase_f(acc_ref, hs2_ref, dpt_ref, b2_ref, g2_ref, be2_ref, out_ref):
  i = pl.program_id(0)
  dinv = _dinv_of(dpt_ref[...])
  o = [dinv * (acc_ref[cc] + hs2_ref[cc]) + b2_ref[0, cc * 32:(cc + 1) * 32]
       for cc in range(4)]
  mu = sum(jnp.sum(oc, axis=1, keepdims=True) for oc in o) * (1.0 / 128.0)
  d = [oc - mu for oc in o]
  var = sum(jnp.sum(dc * dc, axis=1, keepdims=True) for dc in d) * (1.0 / 128.0)
  rstd = lax.rsqrt(var + _EPS)
  parts = []
  for cc in range(4):
    sl = slice(cc * 32, (cc + 1) * 32)
    z = d[cc] * rstd * g2_ref[0, sl] + be2_ref[0, sl]
    parts.append(jnp.sum(z, axis=0, keepdims=True))
  p = jnp.concatenate(parts, axis=0)

  @pl.when(i == 0)
  def _():
    out_ref[...] = jnp.zeros((4, 32), jnp.float32)

  out_ref[...] += p

  @pl.when(i == NBLK - 1)
  def _():
    out_ref[...] = out_ref[...] * (1.0 / N)


@jax.jit
def kernel(x, edge_index, W1, b1, ln1_w, ln1_b, W2, b2, ln2_w, ln2_b):
  src = edge_index[0].astype(jnp.int32)
  dst = edge_index[1].astype(jnp.int32)
  pad = EP - E
  pad_idx = jnp.arange(pad, dtype=jnp.int32)
  src2 = jnp.concatenate([src, pad_idx % 1024]).reshape(EROWS, 128)
  dst2 = jnp.concatenate([dst, N + 16 + pad_idx % 128]).reshape(EROWS, 128)

  deg_parts = _deg_kernel(dst2).reshape(2, NP)
  dpt = jnp.swapaxes(deg_parts, 0, 1)[:N]

  b1r, g1r, be1r = b1.reshape(1, 128), ln1_w.reshape(1, 128), ln1_b.reshape(1, 128)
  b2r, g2r, be2r = b2.reshape(1, 128), ln2_w.reshape(1, 128), ln2_b.reshape(1, 128)

  hs1 = pl.pallas_call(
      _phase_b,
      grid=(NBLK,),
      in_specs=[
          pl.BlockSpec((BN, D_IN), lambda i: (i, 0)),
          pl.BlockSpec((D_IN, D_HID), lambda i: (0, 0)),
          pl.BlockSpec((BN, 2), lambda i: (i, 0)),
      ],
      out_specs=pl.BlockSpec((4, BN, 32), lambda i: (0, i, 0)),
      out_shape=jax.ShapeDtypeStruct((4, N, 32), jnp.float32),
  )(x, W1, dpt)

  acc1 = _scatter_kernel(src2, dst2, hs1.reshape(4 * N, 32))

  hs2 = pl.pallas_call(
      _phase_d,
      grid=(NBLK,),
      in_specs=[
          pl.BlockSpec((4, BN, 32), lambda i: (0, i, 0)),
          pl.BlockSpec((4, BN, 32), lambda i: (0, i, 0)),
          pl.BlockSpec((BN, 2), lambda i: (i, 0)),
          pl.BlockSpec((1, D_HID), lambda i: (0, 0)),
          pl.BlockSpec((1, D_HID), lambda i: (0, 0)),
          pl.BlockSpec((1, D_HID), lambda i: (0, 0)),
          pl.BlockSpec((D_HID, D_HID), lambda i: (0, 0)),
      ],
      out_specs=pl.BlockSpec((4, BN, 32), lambda i: (0, i, 0)),
      out_shape=jax.ShapeDtypeStruct((4, N, 32), jnp.float32),
  )(acc1, hs1, dpt, b1r, g1r, be1r, W2)

  acc2 = _scatter_kernel(src2, dst2, hs2.reshape(4 * N, 32))

  m = pl.pallas_call(
      _phase_f,
      grid=(NBLK,),
      in_specs=[
          pl.BlockSpec((4, BN, 32), lambda i: (0, i, 0)),
          pl.BlockSpec((4, BN, 32), lambda i: (0, i, 0)),
          pl.BlockSpec((BN, 2), lambda i: (i, 0)),
          pl.BlockSpec((1, D_HID), lambda i: (0, 0)),
          pl.BlockSpec((1, D_HID), lambda i: (0, 0)),
          pl.BlockSpec((1, D_HID), lambda i: (0, 0)),
      ],
      out_specs=pl.BlockSpec((4, 32), lambda i: (0, 0)),
      out_shape=jax.ShapeDtypeStruct((4, 32), jnp.float32),
  )(acc2, hs2, dpt, b2r, g2r, be2r)

  return m.reshape(1, 128)


# speedup vs baseline: 14.5922x; 1.3156x over previous
import functools

import jax
import jax.numpy as jnp
from jax import lax
from jax.experimental import pallas as pl
from jax.experimental.pallas import tpu as pltpu
from jax.experimental.pallas import tpu_sc as plsc

N = 50000
E = 800000
D_IN = 64
D_HID = 128

NP = 50176
TPT = NP // 16
EP = 819200
EROWS = EP // 128
RPT = EROWS // 16
KB = 40
NB_OUT = RPT // KB
DROWS = EROWS // 2
DRPT = DROWS // 16
ZR = 784

_mesh = plsc.VectorSubcoreMesh(core_axis_name="c", subcore_axis_name="s")


def _fill(ref, rows, val):
  v = jnp.full((16,), val, jnp.float32)
  if len(ref.shape) == 1:
    @pl.loop(0, rows // 16)
    def _(i):
      ref[pl.ds(i * 16, 16)] = v
  else:
    @pl.loop(0, rows)
    def _(i):
      ref[i, 0:16] = v
      ref[i, 16:32] = v


@functools.partial(
    pl.kernel,
    out_type=jax.ShapeDtypeStruct((2 * NP,), jnp.float32),
    mesh=_mesh,
    scratch_types=[
        pltpu.VMEM((KB, 128), jnp.int32),
        pltpu.VMEM((128,), jnp.float32),
        pltpu.VMEM((ZR,), jnp.float32),
        pltpu.VMEM((TPT,), jnp.float32),
        pltpu.VMEM_SHARED((NP,), jnp.float32),
    ],
)
def _deg_kernel(dst_hbm, deg_out, idx_d, ones_v, zb, fb, deg_sp):
  c = lax.axis_index("c")
  s = lax.axis_index("s")
  _fill(ones_v, 128, 1.0)
  _fill(zb, ZR, 0.0)
  for z in range(4):
    pltpu.sync_copy(zb, deg_sp.at[pl.ds(s * TPT + z * ZR, ZR)])
  plsc.subcore_barrier()

  @pl.loop(0, DRPT // KB)
  def _(kb):
    br = c * DROWS + s * DRPT + kb * KB
    pltpu.sync_copy(dst_hbm.at[pl.ds(br, KB)], idx_d)

    @pl.loop(0, KB)
    def _(j):
      pltpu.sync_copy(ones_v, deg_sp.at[idx_d.at[j]], add=True)

  plsc.subcore_barrier()
  pltpu.sync_copy(deg_sp.at[pl.ds(s * TPT, TPT)], fb)
  pltpu.sync_copy(fb, deg_out.at[pl.ds(c * NP + s * TPT, TPT)])


@functools.partial(
    pl.kernel,
    out_type=jax.ShapeDtypeStruct((4, NP, 32), jnp.float32),
    mesh=_mesh,
    scratch_types=[
        pltpu.VMEM((KB, 128), jnp.int32),
        pltpu.VMEM((KB, 128), jnp.int32),
        pltpu.VMEM((4, 128, 32), jnp.float32),
        pltpu.VMEM((98, 32), jnp.float32),
        pltpu.VMEM_SHARED((NP, 32), jnp.float32),
        pltpu.SemaphoreType.DMA,
        pltpu.SemaphoreType.DMA,
    ],
    compiler_params=pltpu.CompilerParams(use_tc_tiling_on_sc=False),
)
def _scatter_kernel(src_hbm, dst_hbm, hs_hbm, acc_out,
                    idx_s, idx_d, rows, zf, acc_sp, sem, sem2):
  c = lax.axis_index("c")
  s = lax.axis_index("s")
  zrows = 98
  for p in range(2):
    chunk = c * 2 + p
    off = chunk * N
    _fill(zf, zrows, 0.0)
    for z in range(32):
      pltpu.sync_copy(zf, acc_sp.at[pl.ds(s * TPT + z * zrows, zrows)])
    plsc.subcore_barrier()

    @pl.loop(0, NB_OUT)
    def _(kb):
      br = s * RPT + kb * KB
      pltpu.sync_copy(src_hbm.at[pl.ds(br, KB)], idx_s)
      pltpu.sync_copy(dst_hbm.at[pl.ds(br, KB)], idx_d)

      @pl.loop(0, KB)
      def _(j):
        for m in range(8):
          idx_s[j, pl.ds(m * 16, 16)] = idx_s[j, pl.ds(m * 16, 16)] + off

      pltpu.async_copy(hs_hbm.at[idx_s.at[0]], rows.at[0], sem)
      pltpu.async_copy(hs_hbm.at[idx_s.at[1]], rows.at[1], sem)

      @pl.loop(0, KB - 2)
      def _(j):
        par = lax.rem(j, 4)
        pltpu.make_async_copy(hs_hbm.at[idx_s.at[j]], rows.at[par], sem).wait()
        pltpu.async_copy(hs_hbm.at[idx_s.at[j + 2]],
                         rows.at[lax.rem(j + 2, 4)], sem)
        pltpu.async_copy(rows.at[par], acc_sp.at[idx_d.at[j]], sem2, add=True)

        @pl.when(j >= 1)
        def _():
          pm = lax.rem(j - 1, 4)
          pltpu.make_async_copy(rows.at[pm], acc_sp.at[idx_d.at[j - 1]],
                                sem2).wait()

      for j in (KB - 2, KB - 1):
        par = j % 4
        pltpu.make_async_copy(hs_hbm.at[idx_s.at[j]], rows.at[par], sem).wait()
        pltpu.async_copy(rows.at[par], acc_sp.at[idx_d.at[j]], sem2, add=True)
        pm = (j - 1) % 4
        pltpu.make_async_copy(rows.at[pm], acc_sp.at[idx_d.at[j - 1]],
                              sem2).wait()
      pltpu.make_async_copy(rows.at[(KB - 1) % 4],
                            acc_sp.at[idx_d.at[KB - 1]], sem2).wait()

    plsc.subcore_barrier()
    for z in range(32):
      pltpu.sync_copy(acc_sp.at[pl.ds(s * TPT + z * zrows, zrows)], zf)
      pltpu.sync_copy(zf, acc_out.at[chunk, pl.ds(s * TPT + z * zrows, zrows)])
    plsc.subcore_barrier()


BN = 1000
NBLK = N // BN
_EPS = 1e-5


def _dinv_of(dpt_blk):
  deg = dpt_blk[:, 0:1] + dpt_blk[:, 1:2] + 1.0
  return lax.rsqrt(deg)


def _phase_b(x_ref, w1_ref, dpt_ref, out_ref):
  h = jnp.dot(x_ref[...], w1_ref[...], preferred_element_type=jnp.float32,
              precision=lax.Precision.HIGHEST)
  hs = h * _dinv_of(dpt_ref[...])
  for cc in range(4):
    out_ref[cc] = hs[:, cc * 32:(cc + 1) * 32]


def _phase_d(acc_ref, hs1_ref, dpt_ref, b1_ref, g1_ref, be1_ref, w2_ref,
             out_ref):
  dinv = _dinv_of(dpt_ref[...])
  o = [dinv * (acc_ref[cc] + hs1_ref[cc]) + b1_ref[0, cc * 32:(cc + 1) * 32]
       for cc in range(4)]
  mu = sum(jnp.sum(oc, axis=1, keepdims=True) for oc in o) * (1.0 / 128.0)
  d = [oc - mu for oc in o]
  var = sum(jnp.sum(dc * dc, axis=1, keepdims=True) for dc in d) * (1.0 / 128.0)
  rstd = lax.rsqrt(var + _EPS)
  h2 = None
  for cc in range(4):
    sl = slice(cc * 32, (cc + 1) * 32)
    y = jnp.maximum(d[cc] * rstd * g1_ref[0, sl] + be1_ref[0, sl], 0.0)
    t = jnp.dot(y, w2_ref[sl, :], preferred_element_type=jnp.float32,
                precision=lax.Precision.HIGHEST)
    h2 = t if h2 is None else h2 + t
  hs2 = h2 * dinv
  for cc in range(4):
    out_ref[cc] = hs2[:, cc * 32:(cc + 1) * 32]


def _phase_f(acc_ref, hs2_ref, dpt_ref, b2_ref, g2_ref, be2_ref, out_ref):
  i = pl.program_id(0)
  dinv = _dinv_of(dpt_ref[...])
  o = [dinv * (acc_ref[cc] + hs2_ref[cc]) + b2_ref[0, cc * 32:(cc + 1) * 32]
       for cc in range(4)]
  mu = sum(jnp.sum(oc, axis=1, keepdims=True) for oc in o) * (1.0 / 128.0)
  d = [oc - mu for oc in o]
  var = sum(jnp.sum(dc * dc, axis=1, keepdims=True) for dc in d) * (1.0 / 128.0)
  rstd = lax.rsqrt(var + _EPS)
  parts = []
  for cc in range(4):
    sl = slice(cc * 32, (cc + 1) * 32)
    z = d[cc] * rstd * g2_ref[0, sl] + be2_ref[0, sl]
    parts.append(jnp.sum(z, axis=0, keepdims=True))
  p = jnp.concatenate(parts, axis=0)

  @pl.when(i == 0)
  def _():
    out_ref[...] = jnp.zeros((4, 32), jnp.float32)

  out_ref[...] += p

  @pl.when(i == NBLK - 1)
  def _():
    out_ref[...] = out_ref[...] * (1.0 / N)


@jax.jit
def kernel(x, edge_index, W1, b1, ln1_w, ln1_b, W2, b2, ln2_w, ln2_b):
  src = edge_index[0].astype(jnp.int32)
  dst = edge_index[1].astype(jnp.int32)
  pad = EP - E
  pad_idx = jnp.arange(pad, dtype=jnp.int32)
  src2 = jnp.concatenate([src, pad_idx % 1024]).reshape(EROWS, 128)
  dst2 = jnp.concatenate([dst, N + 16 + pad_idx % 128]).reshape(EROWS, 128)

  deg_parts = _deg_kernel(dst2).reshape(2, NP)
  dpt = jnp.swapaxes(deg_parts, 0, 1)[:N]

  b1r, g1r, be1r = b1.reshape(1, 128), ln1_w.reshape(1, 128), ln1_b.reshape(1, 128)
  b2r, g2r, be2r = b2.reshape(1, 128), ln2_w.reshape(1, 128), ln2_b.reshape(1, 128)

  hs1 = pl.pallas_call(
      _phase_b,
      grid=(NBLK,),
      in_specs=[
          pl.BlockSpec((BN, D_IN), lambda i: (i, 0)),
          pl.BlockSpec((D_IN, D_HID), lambda i: (0, 0)),
          pl.BlockSpec((BN, 2), lambda i: (i, 0)),
      ],
      out_specs=pl.BlockSpec((4, BN, 32), lambda i: (0, i, 0)),
      out_shape=jax.ShapeDtypeStruct((4, N, 32), jnp.float32),
  )(x, W1, dpt)

  acc1 = _scatter_kernel(src2, dst2, hs1.reshape(4 * N, 32))

  hs2 = pl.pallas_call(
      _phase_d,
      grid=(NBLK,),
      in_specs=[
          pl.BlockSpec((4, BN, 32), lambda i: (0, i, 0)),
          pl.BlockSpec((4, BN, 32), lambda i: (0, i, 0)),
          pl.BlockSpec((BN, 2), lambda i: (i, 0)),
          pl.BlockSpec((1, D_HID), lambda i: (0, 0)),
          pl.BlockSpec((1, D_HID), lambda i: (0, 0)),
          pl.BlockSpec((1, D_HID), lambda i: (0, 0)),
          pl.BlockSpec((D_HID, D_HID), lambda i: (0, 0)),
      ],
      out_specs=pl.BlockSpec((4, BN, 32), lambda i: (0, i, 0)),
      out_shape=jax.ShapeDtypeStruct((4, N, 32), jnp.float32),
  )(acc1, hs1, dpt, b1r, g1r, be1r, W2)

  acc2 = _scatter_kernel(src2, dst2, hs2.reshape(4 * N, 32))

  m = pl.pallas_call(
      _phase_f,
      grid=(NBLK,),
      in_specs=[
          pl.BlockSpec((4, BN, 32), lambda i: (0, i, 0)),
          pl.BlockSpec((4, BN, 32), lambda i: (0, i, 0)),
          pl.BlockSpec((BN, 2), lambda i: (i, 0)),
          pl.BlockSpec((1, D_HID), lambda i: (0, 0)),
          pl.BlockSpec((1, D_HID), lambda i: (0, 0)),
          pl.BlockSpec((1, D_HID), lambda i: (0, 0)),
      ],
      out_specs=pl.BlockSpec((4, 32), lambda i: (0, 0)),
      out_shape=jax.ShapeDtypeStruct((4, 32), jnp.float32),
  )(acc2, hs2, dpt, b2r, g2r, be2r)

  return m.reshape(1, 128)


# speedup vs baseline: 20.9254x; 1.4340x over previous
import functools

import jax
import jax.numpy as jnp
from jax import lax
from jax.experimental import pallas as pl
from jax.experimental.pallas import tpu as pltpu
from jax.experimental.pallas import tpu_sc as plsc

N = 50000
E = 800000
D_IN = 64
D_HID = 128

NP = 50176
TPT = NP // 16
EP = 819200
EROWS = EP // 128
RPT = EROWS // 16
KB = 40
NB_OUT = RPT // KB
DROWS = EROWS // 2
DRPT = DROWS // 16
ZR = 784

_mesh = plsc.VectorSubcoreMesh(core_axis_name="c", subcore_axis_name="s")


def _fill(ref, rows, val):
  v = jnp.full((16,), val, jnp.float32)
  if len(ref.shape) == 1:
    @pl.loop(0, rows // 16)
    def _(i):
      ref[pl.ds(i * 16, 16)] = v
  else:
    @pl.loop(0, rows)
    def _(i):
      ref[i, 0:16] = v
      ref[i, 16:32] = v


@functools.partial(
    pl.kernel,
    out_type=jax.ShapeDtypeStruct((2 * NP,), jnp.float32),
    mesh=_mesh,
    scratch_types=[
        pltpu.VMEM((KB, 128), jnp.int32),
        pltpu.VMEM((128,), jnp.float32),
        pltpu.VMEM((ZR,), jnp.float32),
        pltpu.VMEM((TPT,), jnp.float32),
        pltpu.VMEM_SHARED((NP,), jnp.float32),
    ],
)
def _deg_kernel(dst_hbm, deg_out, idx_d, ones_v, zb, fb, deg_sp):
  c = lax.axis_index("c")
  s = lax.axis_index("s")
  _fill(ones_v, 128, 1.0)
  _fill(zb, ZR, 0.0)
  for z in range(4):
    pltpu.sync_copy(zb, deg_sp.at[pl.ds(s * TPT + z * ZR, ZR)])
  plsc.subcore_barrier()

  @pl.loop(0, DRPT // KB)
  def _(kb):
    br = c * DROWS + s * DRPT + kb * KB
    pltpu.sync_copy(dst_hbm.at[pl.ds(br, KB)], idx_d)

    @pl.loop(0, KB)
    def _(j):
      pltpu.sync_copy(ones_v, deg_sp.at[idx_d.at[j]], add=True)

  plsc.subcore_barrier()
  pltpu.sync_copy(deg_sp.at[pl.ds(s * TPT, TPT)], fb)
  pltpu.sync_copy(fb, deg_out.at[pl.ds(c * NP + s * TPT, TPT)])


@functools.partial(
    pl.kernel,
    out_type=jax.ShapeDtypeStruct((NP, 128), jnp.float32),
    mesh=_mesh,
    scratch_types=[
        pltpu.VMEM((KB, 128), jnp.int32),
        pltpu.VMEM((KB, 128), jnp.int32),
        pltpu.VMEM((4, 128, 32), jnp.float32),
        pltpu.VMEM((98, 32), jnp.float32),
        pltpu.VMEM_SHARED((NP, 32), jnp.float32),
        pltpu.SemaphoreType.DMA,
        pltpu.SemaphoreType.DMA,
    ],
    compiler_params=pltpu.CompilerParams(use_tc_tiling_on_sc=False),
)
def _scatter_kernel(src_hbm, dst_hbm, hs_hbm, acc_out,
                    idx_s, idx_d, rows, zf, acc_sp, sem, sem2):
  c = lax.axis_index("c")
  s = lax.axis_index("s")
  zrows = 98
  for p in range(2):
    chunk = c * 2 + p
    _fill(zf, zrows, 0.0)
    for z in range(32):
      pltpu.sync_copy(zf, acc_sp.at[pl.ds(s * TPT + z * zrows, zrows)])
    plsc.subcore_barrier()

    @pl.loop(0, NB_OUT)
    def _(kb):
      br = s * RPT + kb * KB
      pltpu.sync_copy(src_hbm.at[pl.ds(br, KB)], idx_s)
      pltpu.sync_copy(dst_hbm.at[pl.ds(br, KB)], idx_d)

      @pl.loop(0, KB)
      def _(j):
        for m in range(8):
          idx_s[j, pl.ds(m * 16, 16)] = idx_s[j, pl.ds(m * 16, 16)] * 4 + chunk

      pltpu.async_copy(hs_hbm.at[idx_s.at[0]], rows.at[0], sem)
      pltpu.async_copy(hs_hbm.at[idx_s.at[1]], rows.at[1], sem)

      @pl.loop(0, KB - 2)
      def _(j):
        par = lax.rem(j, 4)
        pltpu.make_async_copy(hs_hbm.at[idx_s.at[j]], rows.at[par], sem).wait()
        pltpu.async_copy(hs_hbm.at[idx_s.at[j + 2]],
                         rows.at[lax.rem(j + 2, 4)], sem)
        pltpu.async_copy(rows.at[par], acc_sp.at[idx_d.at[j]], sem2, add=True)

        @pl.when(j >= 1)
        def _():
          pm = lax.rem(j - 1, 4)
          pltpu.make_async_copy(rows.at[pm], acc_sp.at[idx_d.at[j - 1]],
                                sem2).wait()

      for j in (KB - 2, KB - 1):
        par = j % 4
        pltpu.make_async_copy(hs_hbm.at[idx_s.at[j]], rows.at[par], sem).wait()
        pltpu.async_copy(rows.at[par], acc_sp.at[idx_d.at[j]], sem2, add=True)
        pm = (j - 1) % 4
        pltpu.make_async_copy(rows.at[pm], acc_sp.at[idx_d.at[j - 1]],
                              sem2).wait()
      pltpu.make_async_copy(rows.at[(KB - 1) % 4],
                            acc_sp.at[idx_d.at[KB - 1]], sem2).wait()

    plsc.subcore_barrier()
    for z in range(32):
      pltpu.sync_copy(acc_sp.at[pl.ds(s * TPT + z * zrows, zrows)], zf)
      pltpu.sync_copy(zf, acc_out.at[pl.ds(s * TPT + z * zrows, zrows),
                                     pl.ds(chunk * 32, 32)])
    plsc.subcore_barrier()


BN = 1000
NBLK = N // BN
_EPS = 1e-5


def _dinv_of(dpt_blk):
  deg = dpt_blk[:, 0:1] + dpt_blk[:, 1:2] + 1.0
  return lax.rsqrt(deg)


def _phase_b(x_ref, w1_ref, dpt_ref, out_ref):
  h = jnp.dot(x_ref[...], w1_ref[...], preferred_element_type=jnp.float32,
              precision=lax.Precision.HIGHEST)
  out_ref[...] = h * _dinv_of(dpt_ref[...])


def _phase_d(acc_ref, hs1_ref, dpt_ref, b1_ref, g1_ref, be1_ref, w2_ref,
             out_ref):
  dinv = _dinv_of(dpt_ref[...])
  o = dinv * (acc_ref[...] + hs1_ref[...]) + b1_ref[0, :]
  mu = jnp.sum(o, axis=1, keepdims=True) * (1.0 / 128.0)
  d = o - mu
  var = jnp.sum(d * d, axis=1, keepdims=True) * (1.0 / 128.0)
  rstd = lax.rsqrt(var + _EPS)
  y = jnp.maximum(d * rstd * g1_ref[0, :] + be1_ref[0, :], 0.0)
  h2 = jnp.dot(y, w2_ref[...], preferred_element_type=jnp.float32,
               precision=lax.Precision.HIGHEST)
  out_ref[...] = h2 * dinv


def _phase_f(acc_ref, hs2_ref, dpt_ref, b2_ref, g2_ref, be2_ref, out_ref):
  i = pl.program_id(0)
  dinv = _dinv_of(dpt_ref[...])
  o = dinv * (acc_ref[...] + hs2_ref[...]) + b2_ref[0, :]
  mu = jnp.sum(o, axis=1, keepdims=True) * (1.0 / 128.0)
  d = o - mu
  var = jnp.sum(d * d, axis=1, keepdims=True) * (1.0 / 128.0)
  rstd = lax.rsqrt(var + _EPS)
  z = d * rstd * g2_ref[0, :] + be2_ref[0, :]
  part = jnp.sum(z, axis=0, keepdims=True)

  @pl.when(i == 0)
  def _():
    out_ref[...] = jnp.zeros((1, 128), jnp.float32)

  out_ref[...] += part

  @pl.when(i == NBLK - 1)
  def _():
    out_ref[...] = out_ref[...] * (1.0 / N)


@jax.jit
def kernel(x, edge_index, W1, b1, ln1_w, ln1_b, W2, b2, ln2_w, ln2_b):
  src = edge_index[0].astype(jnp.int32)
  dst = edge_index[1].astype(jnp.int32)
  pad = EP - E
  pad_idx = jnp.arange(pad, dtype=jnp.int32)
  src2 = jnp.concatenate([src, pad_idx % 1024]).reshape(EROWS, 128)
  dst2 = jnp.concatenate([dst, N + 16 + pad_idx % 128]).reshape(EROWS, 128)

  deg_parts = _deg_kernel(dst2).reshape(2, NP)
  dpt = jnp.swapaxes(deg_parts, 0, 1)[:N]

  b1r, g1r, be1r = b1.reshape(1, 128), ln1_w.reshape(1, 128), ln1_b.reshape(1, 128)
  b2r, g2r, be2r = b2.reshape(1, 128), ln2_w.reshape(1, 128), ln2_b.reshape(1, 128)

  hs1 = pl.pallas_call(
      _phase_b,
      grid=(NBLK,),
      in_specs=[
          pl.BlockSpec((BN, D_IN), lambda i: (i, 0)),
          pl.BlockSpec((D_IN, D_HID), lambda i: (0, 0)),
          pl.BlockSpec((BN, 2), lambda i: (i, 0)),
      ],
      out_specs=pl.BlockSpec((BN, D_HID), lambda i: (i, 0)),
      out_shape=jax.ShapeDtypeStruct((N, D_HID), jnp.float32),
  )(x, W1, dpt)

  acc1 = _scatter_kernel(src2, dst2, hs1.reshape(4 * N, 32))

  hs2 = pl.pallas_call(
      _phase_d,
      grid=(NBLK,),
      in_specs=[
          pl.BlockSpec((BN, D_HID), lambda i: (i, 0)),
          pl.BlockSpec((BN, D_HID), lambda i: (i, 0)),
          pl.BlockSpec((BN, 2), lambda i: (i, 0)),
          pl.BlockSpec((1, D_HID), lambda i: (0, 0)),
          pl.BlockSpec((1, D_HID), lambda i: (0, 0)),
          pl.BlockSpec((1, D_HID), lambda i: (0, 0)),
          pl.BlockSpec((D_HID, D_HID), lambda i: (0, 0)),
      ],
      out_specs=pl.BlockSpec((BN, D_HID), lambda i: (i, 0)),
      out_shape=jax.ShapeDtypeStruct((N, D_HID), jnp.float32),
  )(acc1, hs1, dpt, b1r, g1r, be1r, W2)

  acc2 = _scatter_kernel(src2, dst2, hs2.reshape(4 * N, 32))

  m = pl.pallas_call(
      _phase_f,
      grid=(NBLK,),
      in_specs=[
          pl.BlockSpec((BN, D_HID), lambda i: (i, 0)),
          pl.BlockSpec((BN, D_HID), lambda i: (i, 0)),
          pl.BlockSpec((BN, 2), lambda i: (i, 0)),
          pl.BlockSpec((1, D_HID), lambda i: (0, 0)),
          pl.BlockSpec((1, D_HID), lambda i: (0, 0)),
          pl.BlockSpec((1, D_HID), lambda i: (0, 0)),
      ],
      out_specs=pl.BlockSpec((1, D_HID), lambda i: (0, 0)),
      out_shape=jax.ShapeDtypeStruct((1, D_HID), jnp.float32),
  )(acc2, hs2, dpt, b2r, g2r, be2r)

  return m


# speedup vs baseline: 24.5209x; 1.1718x over previous
import functools

import jax
import jax.numpy as jnp
from jax import lax
from jax.experimental import pallas as pl
from jax.experimental.pallas import tpu as pltpu
from jax.experimental.pallas import tpu_sc as plsc

N = 50000
E = 800000
D_IN = 64
D_HID = 128

NP = 50176
TPT = NP // 16
EP = 819200
EROWS = EP // 128
RPT = EROWS // 16
KB = 40
NB_OUT = RPT // KB
DROWS = EROWS // 2
DRPT = DROWS // 16
ZR = 784

_mesh = plsc.VectorSubcoreMesh(core_axis_name="c", subcore_axis_name="s")


def _fill(ref, rows, val):
  v = jnp.full((16,), val, jnp.float32)
  if len(ref.shape) == 1:
    @pl.loop(0, rows // 16)
    def _(i):
      ref[pl.ds(i * 16, 16)] = v
  else:
    @pl.loop(0, rows)
    def _(i):
      ref[i, 0:16] = v
      ref[i, 16:32] = v


@functools.partial(
    pl.kernel,
    out_type=jax.ShapeDtypeStruct((2 * NP,), jnp.float32),
    mesh=_mesh,
    scratch_types=[
        pltpu.VMEM((KB, 128), jnp.int32),
        pltpu.VMEM((128,), jnp.float32),
        pltpu.VMEM((ZR,), jnp.float32),
        pltpu.VMEM((TPT,), jnp.float32),
        pltpu.VMEM_SHARED((NP,), jnp.float32),
    ],
)
def _deg_kernel(dst_hbm, deg_out, idx_d, ones_v, zb, fb, deg_sp):
  c = lax.axis_index("c")
  s = lax.axis_index("s")
  _fill(ones_v, 128, 1.0)
  _fill(zb, ZR, 0.0)
  for z in range(4):
    pltpu.sync_copy(zb, deg_sp.at[pl.ds(s * TPT + z * ZR, ZR)])
  plsc.subcore_barrier()

  @pl.loop(0, DRPT // KB)
  def _(kb):
    br = c * DROWS + s * DRPT + kb * KB
    pltpu.sync_copy(dst_hbm.at[pl.ds(br, KB)], idx_d)

    @pl.loop(0, KB)
    def _(j):
      pltpu.sync_copy(ones_v, deg_sp.at[idx_d.at[j]], add=True)

  plsc.subcore_barrier()
  pltpu.sync_copy(deg_sp.at[pl.ds(s * TPT, TPT)], fb)
  pltpu.sync_copy(fb, deg_out.at[pl.ds(c * NP + s * TPT, TPT)])


@functools.partial(
    pl.kernel,
    out_type=jax.ShapeDtypeStruct((NP, 128), jnp.float32),
    mesh=_mesh,
    scratch_types=[
        pltpu.VMEM((KB, 128), jnp.int32),
        pltpu.VMEM((KB, 128), jnp.int32),
        pltpu.VMEM((4, 128, 32), jnp.float32),
        pltpu.VMEM((98, 32), jnp.float32),
        pltpu.VMEM_SHARED((NP, 32), jnp.float32),
        pltpu.SemaphoreType.DMA,
        pltpu.SemaphoreType.DMA,
    ],
    compiler_params=pltpu.CompilerParams(use_tc_tiling_on_sc=False),
)
def _scatter_kernel(src_hbm, dst_hbm, hs_hbm, acc_out,
                    idx_s, idx_d, rows, zf, acc_sp, sem, sem2):
  c = lax.axis_index("c")
  s = lax.axis_index("s")
  zrows = 98
  for p in range(2):
    chunk = c * 2 + p
    _fill(zf, zrows, 0.0)
    for z in range(32):
      pltpu.sync_copy(zf, acc_sp.at[pl.ds(s * TPT + z * zrows, zrows)])
    plsc.subcore_barrier()

    @pl.loop(0, NB_OUT)
    def _(kb):
      br = s * RPT + kb * KB
      pltpu.sync_copy(src_hbm.at[pl.ds(br, KB)], idx_s)
      pltpu.sync_copy(dst_hbm.at[pl.ds(br, KB)], idx_d)

      @pl.loop(0, KB)
      def _(j):
        for m in range(8):
          idx_s[j, pl.ds(m * 16, 16)] = idx_s[j, pl.ds(m * 16, 16)] * 4 + chunk

      pltpu.async_copy(hs_hbm.at[idx_s.at[0]], rows.at[0], sem)
      pltpu.async_copy(hs_hbm.at[idx_s.at[1]], rows.at[1], sem)
      pltpu.async_copy(hs_hbm.at[idx_s.at[2]], rows.at[2], sem)

      @pl.loop(0, KB - 3)
      def _(j):
        par = lax.rem(j, 4)
        pltpu.make_async_copy(hs_hbm.at[idx_s.at[j]], rows.at[par], sem).wait()
        pltpu.async_copy(rows.at[par], acc_sp.at[idx_d.at[j]], sem2, add=True)

        @pl.when(j >= 1)
        def _():
          pm = lax.rem(j - 1, 4)
          pltpu.make_async_copy(rows.at[pm], acc_sp.at[idx_d.at[j - 1]],
                                sem2).wait()

        pltpu.async_copy(hs_hbm.at[idx_s.at[j + 3]],
                         rows.at[lax.rem(j + 3, 4)], sem)

      for j in (KB - 3, KB - 2, KB - 1):
        par = j % 4
        pltpu.make_async_copy(hs_hbm.at[idx_s.at[j]], rows.at[par], sem).wait()
        pltpu.async_copy(rows.at[par], acc_sp.at[idx_d.at[j]], sem2, add=True)
        pm = (j - 1) % 4
        pltpu.make_async_copy(rows.at[pm], acc_sp.at[idx_d.at[j - 1]],
                              sem2).wait()
      pltpu.make_async_copy(rows.at[(KB - 1) % 4],
                            acc_sp.at[idx_d.at[KB - 1]], sem2).wait()

    plsc.subcore_barrier()
    for z in range(32):
      pltpu.sync_copy(acc_sp.at[pl.ds(s * TPT + z * zrows, zrows)], zf)
      pltpu.sync_copy(zf, acc_out.at[pl.ds(s * TPT + z * zrows, zrows),
                                     pl.ds(chunk * 32, 32)])
    plsc.subcore_barrier()


BN = 1000
NBLK = N // BN
_EPS = 1e-5


def _dinv_of(dpt_blk):
  deg = dpt_blk[:, 0:1] + dpt_blk[:, 1:2] + 1.0
  return lax.rsqrt(deg)


def _phase_b(x_ref, w1_ref, dpt_ref, out_ref):
  h = jnp.dot(x_ref[...], w1_ref[...], preferred_element_type=jnp.float32,
              precision=lax.Precision.HIGHEST)
  out_ref[...] = h * _dinv_of(dpt_ref[...])


def _phase_d(acc_ref, hs1_ref, dpt_ref, b1_ref, g1_ref, be1_ref, w2_ref,
             out_ref):
  dinv = _dinv_of(dpt_ref[...])
  o = dinv * (acc_ref[...] + hs1_ref[...]) + b1_ref[0, :]
  mu = jnp.sum(o, axis=1, keepdims=True) * (1.0 / 128.0)
  d = o - mu
  var = jnp.sum(d * d, axis=1, keepdims=True) * (1.0 / 128.0)
  rstd = lax.rsqrt(var + _EPS)
  y = jnp.maximum(d * rstd * g1_ref[0, :] + be1_ref[0, :], 0.0)
  h2 = jnp.dot(y, w2_ref[...], preferred_element_type=jnp.float32,
               precision=lax.Precision.HIGHEST)
  out_ref[...] = h2 * dinv


def _phase_f(acc_ref, hs2_ref, dpt_ref, b2_ref, g2_ref, be2_ref, out_ref):
  i = pl.program_id(0)
  dinv = _dinv_of(dpt_ref[...])
  o = dinv * (acc_ref[...] + hs2_ref[...]) + b2_ref[0, :]
  mu = jnp.sum(o, axis=1, keepdims=True) * (1.0 / 128.0)
  d = o - mu
  var = jnp.sum(d * d, axis=1, keepdims=True) * (1.0 / 128.0)
  rstd = lax.rsqrt(var + _EPS)
  z = d * rstd * g2_ref[0, :] + be2_ref[0, :]
  part = jnp.sum(z, axis=0, keepdims=True)

  @pl.when(i == 0)
  def _():
    out_ref[...] = jnp.zeros((1, 128), jnp.float32)

  out_ref[...] += part

  @pl.when(i == NBLK - 1)
  def _():
    out_ref[...] = out_ref[...] * (1.0 / N)


@jax.jit
def kernel(x, edge_index, W1, b1, ln1_w, ln1_b, W2, b2, ln2_w, ln2_b):
  src = edge_index[0].astype(jnp.int32)
  dst = edge_index[1].astype(jnp.int32)
  pad = EP - E
  pad_idx = jnp.arange(pad, dtype=jnp.int32)
  src2 = jnp.concatenate([src, pad_idx % 1024]).reshape(EROWS, 128)
  dst2 = jnp.concatenate([dst, N + 16 + pad_idx % 128]).reshape(EROWS, 128)

  deg_parts = _deg_kernel(dst2).reshape(2, NP)
  dpt = jnp.swapaxes(deg_parts, 0, 1)[:N]

  b1r, g1r, be1r = b1.reshape(1, 128), ln1_w.reshape(1, 128), ln1_b.reshape(1, 128)
  b2r, g2r, be2r = b2.reshape(1, 128), ln2_w.reshape(1, 128), ln2_b.reshape(1, 128)

  hs1 = pl.pallas_call(
      _phase_b,
      grid=(NBLK,),
      in_specs=[
          pl.BlockSpec((BN, D_IN), lambda i: (i, 0)),
          pl.BlockSpec((D_IN, D_HID), lambda i: (0, 0)),
          pl.BlockSpec((BN, 2), lambda i: (i, 0)),
      ],
      out_specs=pl.BlockSpec((BN, D_HID), lambda i: (i, 0)),
      out_shape=jax.ShapeDtypeStruct((N, D_HID), jnp.float32),
  )(x, W1, dpt)

  acc1 = _scatter_kernel(src2, dst2, hs1.reshape(4 * N, 32))

  hs2 = pl.pallas_call(
      _phase_d,
      grid=(NBLK,),
      in_specs=[
          pl.BlockSpec((BN, D_HID), lambda i: (i, 0)),
          pl.BlockSpec((BN, D_HID), lambda i: (i, 0)),
          pl.BlockSpec((BN, 2), lambda i: (i, 0)),
          pl.BlockSpec((1, D_HID), lambda i: (0, 0)),
          pl.BlockSpec((1, D_HID), lambda i: (0, 0)),
          pl.BlockSpec((1, D_HID), lambda i: (0, 0)),
          pl.BlockSpec((D_HID, D_HID), lambda i: (0, 0)),
      ],
      out_specs=pl.BlockSpec((BN, D_HID), lambda i: (i, 0)),
      out_shape=jax.ShapeDtypeStruct((N, D_HID), jnp.float32),
  )(acc1, hs1, dpt, b1r, g1r, be1r, W2)

  acc2 = _scatter_kernel(src2, dst2, hs2.reshape(4 * N, 32))

  m = pl.pallas_call(
      _phase_f,
      grid=(NBLK,),
      in_specs=[
          pl.BlockSpec((BN, D_HID), lambda i: (i, 0)),
          pl.BlockSpec((BN, D_HID), lambda i: (i, 0)),
          pl.BlockSpec((BN, 2), lambda i: (i, 0)),
          pl.BlockSpec((1, D_HID), lambda i: (0, 0)),
          pl.BlockSpec((1, D_HID), lambda i: (0, 0)),
          pl.BlockSpec((1, D_HID), lambda i: (0, 0)),
      ],
      out_specs=pl.BlockSpec((1, D_HID), lambda i: (0, 0)),
      out_shape=jax.ShapeDtypeStruct((1, D_HID), jnp.float32),
  )(acc2, hs2, dpt, b2r, g2r, be2r)

  return m
